# baseline (speedup 1.0000x reference)
import functools

import jax
import jax.numpy as jnp
from jax import lax
from jax.experimental import pallas as pl
from jax.experimental.pallas import tpu as pltpu
from jax.experimental.pallas import tpu_sc as plsc

N_NODES = 10000
N_NETS = 10000
N_SITES = 10000
EMB = 128
BR = 1000


def _leaky(x):
    return jnp.maximum(x, 0.01 * x)


def _ln(y, g, b):
    m = jnp.mean(y, axis=-1, keepdims=True)
    v = jnp.mean((y - m) ** 2, axis=-1, keepdims=True)
    return (y - m) * jax.lax.rsqrt(v + 1e-5) * g + b


def _dot(a, b):
    return jnp.dot(a, b, preferred_element_type=jnp.float32)


def _mlp2_body(nx, want_min, has_add, has_mn, *refs):
    i = pl.program_id(0)
    idx = 0
    xs = refs[idx:idx + nx]; idx += nx
    w1s = refs[idx:idx + nx]; idx += nx
    b1, w2, b2 = refs[idx:idx + 3]; idx += 3
    add_ref = mn_ref_in = None
    if has_add:
        add_ref = refs[idx]; idx += 1
    if has_mn:
        mn_ref_in = refs[idx]; idx += 1
    out_ref = refs[idx]; idx += 1
    h = _dot(xs[0][...], w1s[0][...])
    for k in range(1, nx):
        xk = xs[k][...]
        if has_mn and k == 1:
            xk = jnp.maximum(xk, mn_ref_in[...])
        h = h + _dot(xk, w1s[k][...])
    h = _leaky(h + b1[...])
    y = _dot(h, w2[...]) + b2[...]
    if has_add:
        y = y + add_ref[...]
    out_ref[...] = y
    if want_min:
        mn_ref = refs[idx]
        @pl.when(i == 0)
        def _():
            mn_ref[...] = jnp.full((1, 1), 3.4e38, jnp.float32)
        blk_min = jnp.min(xs[0][...], keepdims=True).reshape(1, 1)
        mn_ref[...] = jnp.minimum(mn_ref[...], blk_min)


def _pl_mlp2(xs, w1s, b1, w2, b2, add=None, min_of_in0=False, mn=None):
    R = xs[0].shape[0]
    n_out = w2.shape[1]
    grid = (R // BR,)
    in_specs = (
        [pl.BlockSpec((BR, x.shape[1]), lambda i: (i, 0)) for x in xs]
        + [pl.BlockSpec(w.shape, lambda i: (0, 0)) for w in w1s]
        + [pl.BlockSpec((1, b1.shape[0]), lambda i: (0, 0)),
           pl.BlockSpec(w2.shape, lambda i: (0, 0)),
           pl.BlockSpec((1, n_out), lambda i: (0, 0))]
    )
    ops = [xs[k] for k in range(len(xs))] + list(w1s) + [
        b1.reshape(1, -1), w2, b2.reshape(1, -1)]
    if add is not None:
        in_specs.append(pl.BlockSpec((BR, n_out), lambda i: (i, 0)))
        ops.append(add)
    if mn is not None:
        in_specs.append(pl.BlockSpec((1, 1), lambda i: (0, 0)))
        ops.append(mn.reshape(1, 1))
    out_shape = [jax.ShapeDtypeStruct((R, n_out), jnp.float32)]
    out_specs = [pl.BlockSpec((BR, n_out), lambda i: (i, 0))]
    if min_of_in0:
        out_shape.append(jax.ShapeDtypeStruct((1, 1), jnp.float32))
        out_specs.append(pl.BlockSpec((1, 1), lambda i: (0, 0)))
    fn = pl.pallas_call(
        functools.partial(_mlp2_body, len(xs), min_of_in0, add is not None,
                          mn is not None),
        grid=grid, in_specs=in_specs, out_specs=out_specs,
        out_shape=out_shape)
    res = fn(*ops)
    if min_of_in0:
        return res[0], res[1][0, 0]
    return res[0]


def _backvn_body(x1, x2, wa, wb, b1, w2, b2, wphi, bphi, h2_ref, g_ref):
    h = _leaky(_dot(x1[...], wa[...]) + _dot(x2[...], wb[...]) + b1[...])
    h2 = _dot(h, w2[...]) + b2[...]
    h2_ref[...] = h2
    g_ref[...] = _dot(h2, wphi[...]) + bphi[...]


def _pl_backvn(h_inst, vng, wa, wb, b1, w2, b2, wphi, bphi):
    R = h_inst.shape[0]
    grid = (R // BR,)
    specs_x = [pl.BlockSpec((BR, EMB), lambda i: (i, 0))] * 2
    specs_w = [pl.BlockSpec(w.shape, lambda i: (0, 0))
               for w in (wa, wb, b1.reshape(1, -1), w2, b2.reshape(1, -1),
                         wphi, bphi.reshape(1, -1))]
    fn = pl.pallas_call(
        _backvn_body, grid=grid, in_specs=specs_x + specs_w,
        out_specs=[pl.BlockSpec((BR, EMB), lambda i: (i, 0))] * 2,
        out_shape=[jax.ShapeDtypeStruct((R, EMB), jnp.float32)] * 2)
    return fn(h_inst, vng, wa, wb, b1.reshape(1, -1), w2, b2.reshape(1, -1),
              wphi, bphi.reshape(1, -1))


def _psinet_body(x1, x2a, x2b, x3a, x3b, wa, wb, wc, b, g, bln, hnn_ref, ln_ref):
    hnn = (_dot(x1[...], wa[...]) + _dot(x2a[...] + x2b[...], wb[...])
           + _dot(x3a[...] + x3b[...], wc[...]) + b[...])
    hnn_ref[...] = hnn
    ln_ref[...] = _leaky(_ln(hnn, g[...], bln[...]))


def _pl_psinet(h_net, src_msgs, sink_aggs, wa, wb, wc, b, g, bln):
    R = h_net.shape[0]
    grid = (R // BR,)
    specs_x = [pl.BlockSpec((BR, EMB), lambda i: (i, 0))] * 5
    wops = (wa, wb, wc, b.reshape(1, -1), g.reshape(1, -1), bln.reshape(1, -1))
    specs_w = [pl.BlockSpec(w.shape, lambda i: (0, 0)) for w in wops]
    fn = pl.pallas_call(
        _psinet_body, grid=grid, in_specs=specs_x + specs_w,
        out_specs=[pl.BlockSpec((BR, EMB), lambda i: (i, 0))] * 2,
        out_shape=[jax.ShapeDtypeStruct((R, EMB), jnp.float32)] * 2)
    return fn(h_net, src_msgs[0], src_msgs[1], sink_aggs[0], sink_aggs[1], *wops)


def _psinode_body(x1, x2, x2b, wa, wb, b, g, bln, out_ref, mn_ref):
    i = pl.program_id(0)
    hin = _dot(x1[...], wa[...]) + _dot(x2[...] + x2b[...], wb[...]) + b[...]
    y = _leaky(_ln(hin, g[...], bln[...]))
    out_ref[...] = y
    @pl.when(i == 0)
    def _():
        mn_ref[...] = jnp.full((1, 1), 3.4e38, jnp.float32)
    mn_ref[...] = jnp.minimum(mn_ref[...], jnp.min(y, keepdims=True).reshape(1, 1))


def _pl_psinode(h2, back, back_b, wa, wb, b, g, bln):
    R = h2.shape[0]
    grid = (R // BR,)
    specs_x = [pl.BlockSpec((BR, EMB), lambda i: (i, 0))] * 3
    wops = (wa, wb, b.reshape(1, -1), g.reshape(1, -1), bln.reshape(1, -1))
    specs_w = [pl.BlockSpec(w.shape, lambda i: (0, 0)) for w in wops]
    fn = pl.pallas_call(
        _psinode_body, grid=grid, in_specs=specs_x + specs_w,
        out_specs=[pl.BlockSpec((BR, EMB), lambda i: (i, 0)),
                   pl.BlockSpec((1, 1), lambda i: (0, 0))],
        out_shape=[jax.ShapeDtypeStruct((R, EMB), jnp.float32),
                   jax.ShapeDtypeStruct((1, 1), jnp.float32)])
    out, mn = fn(h2, back, back_b, *wops)
    return out, mn[0, 0]


def _eproj_body(attr, we, out_ref):
    a = attr[...]
    w = we[...]
    acc = a[:, 0:1] * w[0:1, :]
    for k in range(1, 4):
        acc = acc + a[:, k:k + 1] * w[k:k + 1, :]
    out_ref[...] = acc


def _pl_eproj(attr_pad, we):
    R = attr_pad.shape[0]
    BRE = 2048
    fn = pl.pallas_call(
        _eproj_body, grid=(R // BRE,),
        in_specs=[pl.BlockSpec((BRE, 4), lambda i: (i, 0)),
                  pl.BlockSpec((4, EMB), lambda i: (0, 0))],
        out_specs=pl.BlockSpec((BRE, EMB), lambda i: (i, 0)),
        out_shape=jax.ShapeDtypeStruct((R, EMB), jnp.float32))
    return fn(attr_pad, we)


ACC_ROWS = 10240


def _sc_scatter_sum(table, gath2d, scat2d, zeros):
    nchunk = gath2d.shape[0]
    per_w = nchunk // 32
    mesh = plsc.VectorSubcoreMesh(core_axis_name="c", subcore_axis_name="s")

    @functools.partial(
        pl.kernel, mesh=mesh,
        out_type=jax.ShapeDtypeStruct((2, ACC_ROWS, EMB), jnp.float32),
        scratch_types=[
            pltpu.VMEM((1, 128), jnp.int32),
            pltpu.VMEM((1, 128), jnp.int32),
            pltpu.VMEM((128, EMB), jnp.float32),
            pltpu.VMEM_SHARED((ACC_ROWS, EMB), jnp.float32),
            pltpu.SemaphoreType.DMA,
        ])
    def k(table_h, gath_h, scat_h, zeros_h, out_h, gidx, sidx, rows, acc, sem):
        c = lax.axis_index("c")
        s = lax.axis_index("s")

        @pl.when(s == 0)
        def _():
            pltpu.sync_copy(zeros_h, acc)
        plsc.subcore_barrier()

        base = (s * 2 + c) * per_w

        def body(i, carry):
            row = base + i
            pltpu.sync_copy(gath_h.at[pl.ds(row, 1)], gidx)
            pltpu.sync_copy(scat_h.at[pl.ds(row, 1)], sidx)
            pltpu.async_copy(table_h.at[gidx.at[0]], rows, sem).wait()
            pltpu.sync_copy(rows, acc.at[sidx.at[0]], add=True)
            return carry

        lax.fori_loop(0, per_w, body, 0)
        plsc.subcore_barrier()
        stripe = ACC_ROWS // 16
        pltpu.sync_copy(acc.at[pl.ds(s * stripe, stripe)],
                        out_h.at[c].at[pl.ds(s * stripe, stripe)])

    return k(table, gath2d, scat2d, zeros)


def _sc_sink_src(g, h2, ep, sk_g2d, sk_s2d, sr_g2d, sr_s2d, zeros):
    n_sink = sk_g2d.shape[0] // 32
    n_src = sr_g2d.shape[0] // 32
    mesh = plsc.VectorSubcoreMesh(core_axis_name="c", subcore_axis_name="s")

    @functools.partial(
        pl.kernel, mesh=mesh,
        out_type=[jax.ShapeDtypeStruct((2, ACC_ROWS, EMB), jnp.float32),
                  jax.ShapeDtypeStruct((2, ACC_ROWS, EMB), jnp.float32)],
        scratch_types=[
            pltpu.VMEM((1, 128), jnp.int32),
            pltpu.VMEM((1, 128), jnp.int32),
            pltpu.VMEM((128, EMB), jnp.float32),
            pltpu.VMEM_SHARED((ACC_ROWS, EMB), jnp.float32),
            pltpu.SemaphoreType.DMA,
        ])
    def k(g_h, h2_h, ep_h, skg_h, sks_h, srg_h, srs_h, zeros_h,
          sink_o, src_o, gidx, sidx, rows, acc, sem):
        c = lax.axis_index("c")
        s = lax.axis_index("s")
        stripe = ACC_ROWS // 16

        def zero_acc():
            @pl.when(s == 0)
            def _():
                pltpu.sync_copy(zeros_h, acc)

        def leaky_rows(r, carry):
            for j in range(EMB // 16):
                v = rows[r, pl.ds(j * 16, 16)]
                rows[r, pl.ds(j * 16, 16)] = jnp.maximum(v, 0.01 * v)
            return carry

        zero_acc()
        plsc.subcore_barrier()
        base = (s * 2 + c) * n_sink

        def sink_body(i, carry):
            row = base + i
            pltpu.sync_copy(skg_h.at[pl.ds(row, 1)], gidx)
            pltpu.sync_copy(sks_h.at[pl.ds(row, 1)], sidx)
            pltpu.sync_copy(ep_h.at[pl.ds(row * 128, 128)], rows)
            pltpu.sync_copy(g_h.at[gidx.at[0]], rows, add=True)
            lax.fori_loop(0, 128, leaky_rows, 0)
            pltpu.sync_copy(rows, acc.at[sidx.at[0]], add=True)
            return carry

        lax.fori_loop(0, n_sink, sink_body, 0)
        plsc.subcore_barrier()
        pltpu.sync_copy(acc.at[pl.ds(s * stripe, stripe)],
                        sink_o.at[c].at[pl.ds(s * stripe, stripe)])
        plsc.subcore_barrier()
        zero_acc()
        plsc.subcore_barrier()
        base2 = (s * 2 + c) * n_src

        def src_body(i, carry):
            row = base2 + i
            pltpu.sync_copy(srg_h.at[pl.ds(row, 1)], gidx)
            pltpu.sync_copy(srs_h.at[pl.ds(row, 1)], sidx)
            pltpu.sync_copy(h2_h.at[gidx.at[0]], rows)
            pltpu.sync_copy(rows, acc.at[sidx.at[0]], add=True)
            return carry

        lax.fori_loop(0, n_src, src_body, 0)
        plsc.subcore_barrier()
        pltpu.sync_copy(acc.at[pl.ds(s * stripe, stripe)],
                        src_o.at[c].at[pl.ds(s * stripe, stripe)])

    return k(g, h2, ep, sk_g2d, sk_s2d, sr_g2d, sr_s2d, zeros)


def _sc_gather_rows(table, idx_pad):
    n = idx_pad.shape[0]
    per_w = n // 32
    mesh = plsc.VectorSubcoreMesh(core_axis_name="c", subcore_axis_name="s")

    @functools.partial(
        pl.kernel, mesh=mesh,
        out_type=jax.ShapeDtypeStruct((n, EMB), jnp.float32),
        scratch_types=[
            pltpu.VMEM((per_w,), jnp.int32),
            pltpu.VMEM((per_w, EMB), jnp.float32),
            pltpu.SemaphoreType.DMA,
        ])
    def k(table_h, idx_h, out_h, idxv, rows, sem):
        c = lax.axis_index("c")
        s = lax.axis_index("s")
        base = pl.multiple_of((s * 2 + c) * per_w, 8)
        pltpu.sync_copy(idx_h.at[pl.ds(base, per_w)], idxv)
        pltpu.async_copy(table_h.at[idxv], rows, sem).wait()
        pltpu.sync_copy(rows, out_h.at[pl.ds(base, per_w)])

    return k(table, idx_pad)


def _pad_edges_2d(idx, n_pad_to, dummy_base):
    extra = n_pad_to - idx.shape[0]
    if dummy_base is None:
        pad = (jnp.arange(extra, dtype=jnp.int32) * 17) % N_NODES
    else:
        pad = dummy_base + (jnp.arange(extra, dtype=jnp.int32) % 200)
    return jnp.concatenate([idx, pad]).reshape(-1, 128)


def kernel(node_x, net_x, edge_attr_sink_to_net, params,
           edge_index_source_to_net, edge_index_sink_to_net, pos_lst):
    p = params
    idx = pos_lst[:, 0].astype(jnp.int32) * 100 + pos_lst[:, 1].astype(jnp.int32)

    idx_pad = jnp.concatenate([idx, jnp.zeros((240,), jnp.int32)])

    usum = jnp.zeros((N_SITES, EMB), jnp.float32).at[idx].add(node_x)
    umax = jnp.full((N_SITES, EMB), -3.4e38, jnp.float32).at[idx].max(node_x)

    ne = p["node_enc"]
    h_inst, mn0 = _pl_mlp2([node_x], [ne[0]["w"]], ne[0]["b"],
                           ne[1]["w"], ne[1]["b"], min_of_in0=True)
    ve = p["vn_enc"]
    vn = _pl_mlp2([usum, umax],
                  [ve[0]["w"][:EMB], ve[0]["w"][EMB:]], ve[0]["b"],
                  ve[1]["w"], ve[1]["b"], mn=mn0)
    te = p["net_enc"]
    h_net = _pl_mlp2([net_x], [te[0]["w"]], te[0]["b"], te[1]["w"], te[1]["b"])

    sink_src = edge_index_sink_to_net[0]
    sink_dst = edge_index_sink_to_net[1]
    src_src = edge_index_source_to_net[0]
    src_dst = edge_index_source_to_net[1]

    EP = 327680
    EPS = 12288
    sink_dst2d = _pad_edges_2d(sink_dst, EP, None)
    sink_src2d_scat = _pad_edges_2d(sink_src, EP, N_NODES)
    sink_src2d_g = _pad_edges_2d(sink_src, EP, None)
    sink_dst2d_scat = _pad_edges_2d(sink_dst, EP, N_NETS)
    src_src2d_g = _pad_edges_2d(src_src, EPS, None)
    src_dst2d_scat = _pad_edges_2d(src_dst, EPS, N_NETS)
    attr_pad = jnp.pad(edge_attr_sink_to_net, ((0, EP - sink_src.shape[0]), (0, 0)))
    zeros_acc = jnp.zeros((ACC_ROWS, EMB), jnp.float32)

    for lp in p["layers"]:
        vng = _sc_gather_rows(vn, idx_pad)[:N_NODES]
        bv = lp["back_vn"]
        h2, g = _pl_backvn(h_inst, vng,
                           bv[0]["w"][:EMB], bv[0]["w"][EMB:], bv[0]["b"],
                           bv[1]["w"], bv[1]["b"],
                           lp["phi"]["w"][:EMB], lp["phi"]["b"])
        eproj = _pl_eproj(attr_pad, lp["phi"]["w"][EMB:])
        sinkp, srcp = _sc_sink_src(g, h2, eproj,
                                   sink_src2d_g, sink_dst2d_scat,
                                   src_src2d_g, src_dst2d_scat, zeros_acc)

        pn = lp["psi_net"]
        hnn, h_net = _pl_psinet(h_net,
                                (srcp[0, :N_NETS], srcp[1, :N_NETS]),
                                (sinkp[0, :N_NETS], sinkp[1, :N_NETS]),
                                pn["w"][:EMB], pn["w"][EMB:2 * EMB],
                                pn["w"][2 * EMB:], pn["b"],
                                lp["norm_g"], lp["norm_b"])
        backp = _sc_scatter_sum(hnn, sink_dst2d, sink_src2d_scat, zeros_acc)
        pd = lp["psi_node"]
        h_inst, mn = _pl_psinode(h2, backp[0, :N_NODES], backp[1, :N_NODES],
                                 pd["w"][:EMB], pd["w"][EMB:],
                                 pd["b"], lp["norm_g"], lp["norm_b"])

        usum = jnp.zeros((N_SITES, EMB), jnp.float32).at[idx].add(h_inst)
        umax = jnp.full((N_SITES, EMB), -3.4e38, jnp.float32).at[idx].max(h_inst)
        mv = lp["mlp_vn"]
        vn = _pl_mlp2([usum, umax],
                      [mv[0]["w"][:EMB], mv[0]["w"][EMB:]], mv[0]["b"],
                      mv[1]["w"], mv[1]["b"], add=vn, mn=mn)

    def _head(x, l1, l2, nout):
        w2p = jnp.pad(l2["w"], ((0, 0), (0, EMB - nout)))
        b2p = jnp.pad(l2["b"], (0, EMB - nout))
        y = _pl_mlp2([x], [l1["w"]], l1["b"], w2p, b2p)
        return y[:, :nout]

    out_inst = _head(h_inst, p["fc1_node"], p["fc2_node"], 8)
    out_net = _head(h_net, p["fc1_net"], p["fc2_net"], 8)
    out_vn = _head(vn, p["fc1_vn"], p["fc2_vn"], 8)
    return out_inst, out_net, out_vn


# speedup vs baseline: 1.1977x; 1.1977x over previous
import functools

import jax
import jax.numpy as jnp
from jax import lax
from jax.experimental import pallas as pl
from jax.experimental.pallas import tpu as pltpu
from jax.experimental.pallas import tpu_sc as plsc

N_NODES = 10000
N_NETS = 10000
N_SITES = 10000
EMB = 128
BR = 1000


def _leaky(x):
    return jnp.maximum(x, 0.01 * x)


def _ln(y, g, b):
    m = jnp.mean(y, axis=-1, keepdims=True)
    v = jnp.mean((y - m) ** 2, axis=-1, keepdims=True)
    return (y - m) * jax.lax.rsqrt(v + 1e-5) * g + b


def _dot(a, b):
    return jnp.dot(a, b, preferred_element_type=jnp.float32)


def _mlp2_body(nx, want_min, has_add, has_mn, *refs):
    i = pl.program_id(0)
    idx = 0
    xs = refs[idx:idx + nx]; idx += nx
    w1s = refs[idx:idx + nx]; idx += nx
    b1, w2, b2 = refs[idx:idx + 3]; idx += 3
    add_ref = mn_ref_in = None
    if has_add:
        add_ref = refs[idx]; idx += 1
    if has_mn:
        mn_ref_in = refs[idx]; idx += 1
    out_ref = refs[idx]; idx += 1
    h = _dot(xs[0][...], w1s[0][...])
    for k in range(1, nx):
        xk = xs[k][...]
        if has_mn and k == 1:
            xk = jnp.maximum(xk, mn_ref_in[...])
        h = h + _dot(xk, w1s[k][...])
    h = _leaky(h + b1[...])
    y = _dot(h, w2[...]) + b2[...]
    if has_add:
        y = y + add_ref[...]
    out_ref[...] = y
    if want_min:
        mn_ref = refs[idx]
        @pl.when(i == 0)
        def _():
            mn_ref[...] = jnp.full((1, 1), 3.4e38, jnp.float32)
        blk_min = jnp.min(xs[0][...], keepdims=True).reshape(1, 1)
        mn_ref[...] = jnp.minimum(mn_ref[...], blk_min)


def _pl_mlp2(xs, w1s, b1, w2, b2, add=None, min_of_in0=False, mn=None):
    R = xs[0].shape[0]
    n_out = w2.shape[1]
    grid = (R // BR,)
    in_specs = (
        [pl.BlockSpec((BR, x.shape[1]), lambda i: (i, 0)) for x in xs]
        + [pl.BlockSpec(w.shape, lambda i: (0, 0)) for w in w1s]
        + [pl.BlockSpec((1, b1.shape[0]), lambda i: (0, 0)),
           pl.BlockSpec(w2.shape, lambda i: (0, 0)),
           pl.BlockSpec((1, n_out), lambda i: (0, 0))]
    )
    ops = [xs[k] for k in range(len(xs))] + list(w1s) + [
        b1.reshape(1, -1), w2, b2.reshape(1, -1)]
    if add is not None:
        in_specs.append(pl.BlockSpec((BR, n_out), lambda i: (i, 0)))
        ops.append(add)
    if mn is not None:
        in_specs.append(pl.BlockSpec((1, 1), lambda i: (0, 0)))
        ops.append(mn.reshape(1, 1))
    out_shape = [jax.ShapeDtypeStruct((R, n_out), jnp.float32)]
    out_specs = [pl.BlockSpec((BR, n_out), lambda i: (i, 0))]
    if min_of_in0:
        out_shape.append(jax.ShapeDtypeStruct((1, 1), jnp.float32))
        out_specs.append(pl.BlockSpec((1, 1), lambda i: (0, 0)))
    fn = pl.pallas_call(
        functools.partial(_mlp2_body, len(xs), min_of_in0, add is not None,
                          mn is not None),
        grid=grid, in_specs=in_specs, out_specs=out_specs,
        out_shape=out_shape)
    res = fn(*ops)
    if min_of_in0:
        return res[0], res[1][0, 0]
    return res[0]


def _backvn_body(x1, x2, wa, wb, b1, w2, b2, wphi, bphi, h2_ref, g_ref):
    h = _leaky(_dot(x1[...], wa[...]) + _dot(x2[...], wb[...]) + b1[...])
    h2 = _dot(h, w2[...]) + b2[...]
    h2_ref[...] = h2
    g_ref[...] = _dot(h2, wphi[...]) + bphi[...]


def _pl_backvn(h_inst, vng, wa, wb, b1, w2, b2, wphi, bphi):
    R = h_inst.shape[0]
    grid = (R // BR,)
    specs_x = [pl.BlockSpec((BR, EMB), lambda i: (i, 0))] * 2
    specs_w = [pl.BlockSpec(w.shape, lambda i: (0, 0))
               for w in (wa, wb, b1.reshape(1, -1), w2, b2.reshape(1, -1),
                         wphi, bphi.reshape(1, -1))]
    fn = pl.pallas_call(
        _backvn_body, grid=grid, in_specs=specs_x + specs_w,
        out_specs=[pl.BlockSpec((BR, EMB), lambda i: (i, 0))] * 2,
        out_shape=[jax.ShapeDtypeStruct((R, EMB), jnp.float32)] * 2)
    return fn(h_inst, vng, wa, wb, b1.reshape(1, -1), w2, b2.reshape(1, -1),
              wphi, bphi.reshape(1, -1))


def _psinet_body(x1, x2a, x2b, x3a, x3b, wa, wb, wc, b, g, bln, hnn_ref, ln_ref):
    hnn = (_dot(x1[...], wa[...]) + _dot(x2a[...] + x2b[...], wb[...])
           + _dot(x3a[...] + x3b[...], wc[...]) + b[...])
    hnn_ref[...] = hnn
    ln_ref[...] = _leaky(_ln(hnn, g[...], bln[...]))


def _pl_psinet(h_net, src_msgs, sink_aggs, wa, wb, wc, b, g, bln):
    R = h_net.shape[0]
    grid = (R // BR,)
    specs_x = [pl.BlockSpec((BR, EMB), lambda i: (i, 0))] * 5
    wops = (wa, wb, wc, b.reshape(1, -1), g.reshape(1, -1), bln.reshape(1, -1))
    specs_w = [pl.BlockSpec(w.shape, lambda i: (0, 0)) for w in wops]
    fn = pl.pallas_call(
        _psinet_body, grid=grid, in_specs=specs_x + specs_w,
        out_specs=[pl.BlockSpec((BR, EMB), lambda i: (i, 0))] * 2,
        out_shape=[jax.ShapeDtypeStruct((R, EMB), jnp.float32)] * 2)
    return fn(h_net, src_msgs[0], src_msgs[1], sink_aggs[0], sink_aggs[1], *wops)


def _psinode_body(x1, x2, x2b, wa, wb, b, g, bln, out_ref, mn_ref):
    i = pl.program_id(0)
    hin = _dot(x1[...], wa[...]) + _dot(x2[...] + x2b[...], wb[...]) + b[...]
    y = _leaky(_ln(hin, g[...], bln[...]))
    out_ref[...] = y
    @pl.when(i == 0)
    def _():
        mn_ref[...] = jnp.full((1, 1), 3.4e38, jnp.float32)
    mn_ref[...] = jnp.minimum(mn_ref[...], jnp.min(y, keepdims=True).reshape(1, 1))


def _pl_psinode(h2, back, back_b, wa, wb, b, g, bln):
    R = h2.shape[0]
    grid = (R // BR,)
    specs_x = [pl.BlockSpec((BR, EMB), lambda i: (i, 0))] * 3
    wops = (wa, wb, b.reshape(1, -1), g.reshape(1, -1), bln.reshape(1, -1))
    specs_w = [pl.BlockSpec(w.shape, lambda i: (0, 0)) for w in wops]
    fn = pl.pallas_call(
        _psinode_body, grid=grid, in_specs=specs_x + specs_w,
        out_specs=[pl.BlockSpec((BR, EMB), lambda i: (i, 0)),
                   pl.BlockSpec((1, 1), lambda i: (0, 0))],
        out_shape=[jax.ShapeDtypeStruct((R, EMB), jnp.float32),
                   jax.ShapeDtypeStruct((1, 1), jnp.float32)])
    out, mn = fn(h2, back, back_b, *wops)
    return out, mn[0, 0]


def _eproj_body(attr, we, out_ref):
    a = attr[...]
    w = we[...]
    acc = a[:, 0:1] * w[0:1, :]
    for k in range(1, 4):
        acc = acc + a[:, k:k + 1] * w[k:k + 1, :]
    out_ref[...] = acc


def _pl_eproj(attr_pad, we):
    R = attr_pad.shape[0]
    BRE = 2048
    fn = pl.pallas_call(
        _eproj_body, grid=(R // BRE,),
        in_specs=[pl.BlockSpec((BRE, 4), lambda i: (i, 0)),
                  pl.BlockSpec((4, EMB), lambda i: (0, 0))],
        out_specs=pl.BlockSpec((BRE, EMB), lambda i: (i, 0)),
        out_shape=jax.ShapeDtypeStruct((R, EMB), jnp.float32))
    return fn(attr_pad, we)


ACC_ROWS = 10240


def _sc_scatter_sum(table, gath2d, scat2d, zeros):
    nchunk = gath2d.shape[0]
    per_w = nchunk // 32
    mesh = plsc.VectorSubcoreMesh(core_axis_name="c", subcore_axis_name="s")

    @functools.partial(
        pl.kernel, mesh=mesh,
        out_type=jax.ShapeDtypeStruct((2, ACC_ROWS, EMB), jnp.float32),
        scratch_types=[
            pltpu.VMEM((1, 128), jnp.int32),
            pltpu.VMEM((1, 128), jnp.int32),
            pltpu.VMEM((1, 128), jnp.int32),
            pltpu.VMEM((1, 128), jnp.int32),
            pltpu.VMEM((128, EMB), jnp.float32),
            pltpu.VMEM((128, EMB), jnp.float32),
            pltpu.VMEM_SHARED((ACC_ROWS, EMB), jnp.float32),
            pltpu.SemaphoreType.DMA,
            pltpu.SemaphoreType.DMA,
            pltpu.SemaphoreType.DMA,
            pltpu.SemaphoreType.DMA,
            pltpu.SemaphoreType.DMA,
            pltpu.SemaphoreType.DMA,
        ])
    def k(table_h, gath_h, scat_h, zeros_h, out_h, gi0, gi1, si0, si1,
          rows0, rows1, acc, se0, se1, sg0, sg1, ss0, ss1):
        c = lax.axis_index("c")
        s = lax.axis_index("s")

        @pl.when(s == 0)
        def _():
            pltpu.sync_copy(zeros_h, acc)
        base = (s * 2 + c) * per_w
        plsc.subcore_barrier()

        gi = (gi0, gi1)
        si = (si0, si1)
        rows = (rows0, rows1)
        se = (se0, se1)
        sg = (sg0, sg1)
        ss = (ss0, ss1)

        def body(i, carry):
            hi = [None, None]
            hg = [None, None]
            hs = [None, None]
            for b in range(2):
                ch = base + i * 2 + b
                hi[b] = (pltpu.async_copy(gath_h.at[pl.ds(ch, 1)], gi[b],
                                          se[b]),
                         pltpu.async_copy(scat_h.at[pl.ds(ch, 1)], si[b],
                                          se[b]))
            for b in range(2):
                hi[b][0].wait()
                hi[b][1].wait()
                hg[b] = pltpu.async_copy(table_h.at[gi[b].at[0]], rows[b],
                                         sg[b])
            for b in range(2):
                hg[b].wait()
                hs[b] = pltpu.async_copy(rows[b], acc.at[si[b].at[0]],
                                         ss[b], add=True)
            for b in range(2):
                hs[b].wait()
            return carry

        lax.fori_loop(0, per_w // 2, body, 0)
        plsc.subcore_barrier()
        stripe = ACC_ROWS // 16
        pltpu.sync_copy(acc.at[pl.ds(s * stripe, stripe)],
                        out_h.at[c].at[pl.ds(s * stripe, stripe)])

    return k(table, gath2d, scat2d, zeros)


def _sc_sink_src(g, h2, ep, sk_g2d, sk_s2d, sr_g2d, sr_s2d, zeros):
    n_sink = sk_g2d.shape[0] // 32
    n_src = sr_g2d.shape[0] // 32
    mesh = plsc.VectorSubcoreMesh(core_axis_name="c", subcore_axis_name="s")

    @functools.partial(
        pl.kernel, mesh=mesh,
        out_type=[jax.ShapeDtypeStruct((2, ACC_ROWS, EMB), jnp.float32),
                  jax.ShapeDtypeStruct((2, ACC_ROWS, EMB), jnp.float32)],
        scratch_types=[
            pltpu.VMEM((1, 128), jnp.int32),
            pltpu.VMEM((1, 128), jnp.int32),
            pltpu.VMEM((1, 128), jnp.int32),
            pltpu.VMEM((1, 128), jnp.int32),
            pltpu.VMEM((128, EMB), jnp.float32),
            pltpu.VMEM((128, EMB), jnp.float32),
            pltpu.VMEM_SHARED((ACC_ROWS, EMB), jnp.float32),
            pltpu.SemaphoreType.DMA,
            pltpu.SemaphoreType.DMA,
            pltpu.SemaphoreType.DMA,
            pltpu.SemaphoreType.DMA,
            pltpu.SemaphoreType.DMA,
            pltpu.SemaphoreType.DMA,
            pltpu.SemaphoreType.DMA,
            pltpu.SemaphoreType.DMA,
        ])
    def k(g_h, h2_h, ep_h, skg_h, sks_h, srg_h, srs_h, zeros_h,
          sink_o, src_o, gi0, gi1, si0, si1, rows0, rows1, acc,
          si_s0, si_s1, sp0, sp1, sg0, sg1, ss0, ss1):
        c = lax.axis_index("c")
        s = lax.axis_index("s")
        stripe = ACC_ROWS // 16
        gi = (gi0, gi1)
        si = (si0, si1)
        rows = (rows0, rows1)
        sei = (si_s0, si_s1)
        sep = (sp0, sp1)
        sg = (sg0, sg1)
        ss = (ss0, ss1)

        def zero_acc():
            @pl.when(s == 0)
            def _():
                pltpu.sync_copy(zeros_h, acc)

        zero_acc()
        base = (s * 2 + c) * n_sink
        plsc.subcore_barrier()

        def leaky_buf(rbuf):
            def leaky_rows(r, carry):
                for j in range(EMB // 16):
                    v = rbuf[r, pl.ds(j * 16, 16)]
                    rbuf[r, pl.ds(j * 16, 16)] = jnp.maximum(v, 0.01 * v)
                return carry
            lax.fori_loop(0, 128, leaky_rows, 0)

        def sink_body(i, carry):
            hi = [None, None]
            hp = [None, None]
            hg = [None, None]
            hs = [None, None]
            for b in range(2):
                ch = base + i * 2 + b
                hi[b] = (pltpu.async_copy(skg_h.at[pl.ds(ch, 1)], gi[b],
                                          sei[b]),
                         pltpu.async_copy(sks_h.at[pl.ds(ch, 1)], si[b],
                                          sei[b]))
                hp[b] = pltpu.async_copy(
                    ep_h.at[pl.ds(pl.multiple_of(ch * 128, 128), 128)],
                    rows[b], sep[b])
            for b in range(2):
                hi[b][0].wait()
                hi[b][1].wait()
                hp[b].wait()
                hg[b] = pltpu.async_copy(g_h.at[gi[b].at[0]], rows[b],
                                         sg[b], add=True)
            for b in range(2):
                hg[b].wait()
                leaky_buf(rows[b])
                hs[b] = pltpu.async_copy(rows[b], acc.at[si[b].at[0]],
                                         ss[b], add=True)
            for b in range(2):
                hs[b].wait()
            return carry

        lax.fori_loop(0, n_sink // 2, sink_body, 0)
        plsc.subcore_barrier()
        pltpu.sync_copy(acc.at[pl.ds(s * stripe, stripe)],
                        sink_o.at[c].at[pl.ds(s * stripe, stripe)])
        plsc.subcore_barrier()
        zero_acc()
        base2 = (s * 2 + c) * n_src
        plsc.subcore_barrier()

        def src_body(i, carry):
            ch = base2 + i
            pltpu.sync_copy(srg_h.at[pl.ds(ch, 1)], gi0)
            pltpu.sync_copy(srs_h.at[pl.ds(ch, 1)], si0)
            pltpu.async_copy(h2_h.at[gi0.at[0]], rows0, sg0)
            pltpu.make_async_copy(h2_h.at[gi0.at[0]], rows0, sg0).wait()
            pltpu.sync_copy(rows0, acc.at[si0.at[0]], add=True)
            return carry

        lax.fori_loop(0, n_src, src_body, 0)
        plsc.subcore_barrier()
        pltpu.sync_copy(acc.at[pl.ds(s * stripe, stripe)],
                        src_o.at[c].at[pl.ds(s * stripe, stripe)])

    return k(g, h2, ep, sk_g2d, sk_s2d, sr_g2d, sr_s2d, zeros)


def _sc_gather_rows(table, idx_pad):
    n = idx_pad.shape[0]
    per_w = n // 32
    mesh = plsc.VectorSubcoreMesh(core_axis_name="c", subcore_axis_name="s")

    @functools.partial(
        pl.kernel, mesh=mesh,
        out_type=jax.ShapeDtypeStruct((n, EMB), jnp.float32),
        scratch_types=[
            pltpu.VMEM((per_w,), jnp.int32),
            pltpu.VMEM((per_w, EMB), jnp.float32),
            pltpu.SemaphoreType.DMA,
        ])
    def k(table_h, idx_h, out_h, idxv, rows, sem):
        c = lax.axis_index("c")
        s = lax.axis_index("s")
        base = pl.multiple_of((s * 2 + c) * per_w, 8)
        pltpu.sync_copy(idx_h.at[pl.ds(base, per_w)], idxv)
        pltpu.async_copy(table_h.at[idxv], rows, sem).wait()
        pltpu.sync_copy(rows, out_h.at[pl.ds(base, per_w)])

    return k(table, idx_pad)


def _pad_edges_2d(idx, n_pad_to, dummy_base):
    extra = n_pad_to - idx.shape[0]
    if dummy_base is None:
        pad = (jnp.arange(extra, dtype=jnp.int32) * 17) % N_NODES
    else:
        pad = dummy_base + (jnp.arange(extra, dtype=jnp.int32) % 200)
    return jnp.concatenate([idx, pad]).reshape(-1, 128)


def kernel(node_x, net_x, edge_attr_sink_to_net, params,
           edge_index_source_to_net, edge_index_sink_to_net, pos_lst):
    p = params
    idx = pos_lst[:, 0].astype(jnp.int32) * 100 + pos_lst[:, 1].astype(jnp.int32)

    idx_pad = jnp.concatenate([idx, jnp.zeros((240,), jnp.int32)])

    usum = jnp.zeros((N_SITES, EMB), jnp.float32).at[idx].add(node_x)
    umax = jnp.full((N_SITES, EMB), -3.4e38, jnp.float32).at[idx].max(node_x)

    ne = p["node_enc"]
    h_inst, mn0 = _pl_mlp2([node_x], [ne[0]["w"]], ne[0]["b"],
                           ne[1]["w"], ne[1]["b"], min_of_in0=True)
    ve = p["vn_enc"]
    vn = _pl_mlp2([usum, umax],
                  [ve[0]["w"][:EMB], ve[0]["w"][EMB:]], ve[0]["b"],
                  ve[1]["w"], ve[1]["b"], mn=mn0)
    te = p["net_enc"]
    h_net = _pl_mlp2([net_x], [te[0]["w"]], te[0]["b"], te[1]["w"], te[1]["b"])

    sink_src = edge_index_sink_to_net[0]
    sink_dst = edge_index_sink_to_net[1]
    src_src = edge_index_source_to_net[0]
    src_dst = edge_index_source_to_net[1]

    EP = 327680
    EPS = 12288
    sink_dst2d = _pad_edges_2d(sink_dst, EP, None)
    sink_src2d_scat = _pad_edges_2d(sink_src, EP, N_NODES)
    sink_src2d_g = _pad_edges_2d(sink_src, EP, None)
    sink_dst2d_scat = _pad_edges_2d(sink_dst, EP, N_NETS)
    src_src2d_g = _pad_edges_2d(src_src, EPS, None)
    src_dst2d_scat = _pad_edges_2d(src_dst, EPS, N_NETS)
    attr_pad = jnp.pad(edge_attr_sink_to_net, ((0, EP - sink_src.shape[0]), (0, 0)))
    zeros_acc = jnp.zeros((ACC_ROWS, EMB), jnp.float32)

    for lp in p["layers"]:
        vng = _sc_gather_rows(vn, idx_pad)[:N_NODES]
        bv = lp["back_vn"]
        h2, g = _pl_backvn(h_inst, vng,
                           bv[0]["w"][:EMB], bv[0]["w"][EMB:], bv[0]["b"],
                           bv[1]["w"], bv[1]["b"],
                           lp["phi"]["w"][:EMB], lp["phi"]["b"])
        eproj = _pl_eproj(attr_pad, lp["phi"]["w"][EMB:])
        sinkp, srcp = _sc_sink_src(g, h2, eproj,
                                   sink_src2d_g, sink_dst2d_scat,
                                   src_src2d_g, src_dst2d_scat, zeros_acc)

        pn = lp["psi_net"]
        hnn, h_net = _pl_psinet(h_net,
                                (srcp[0, :N_NETS], srcp[1, :N_NETS]),
                                (sinkp[0, :N_NETS], sinkp[1, :N_NETS]),
                                pn["w"][:EMB], pn["w"][EMB:2 * EMB],
                                pn["w"][2 * EMB:], pn["b"],
                                lp["norm_g"], lp["norm_b"])
        backp = _sc_scatter_sum(hnn, sink_dst2d, sink_src2d_scat, zeros_acc)
        pd = lp["psi_node"]
        h_inst, mn = _pl_psinode(h2, backp[0, :N_NODES], backp[1, :N_NODES],
                                 pd["w"][:EMB], pd["w"][EMB:],
                                 pd["b"], lp["norm_g"], lp["norm_b"])

        usum = jnp.zeros((N_SITES, EMB), jnp.float32).at[idx].add(h_inst)
        umax = jnp.full((N_SITES, EMB), -3.4e38, jnp.float32).at[idx].max(h_inst)
        mv = lp["mlp_vn"]
        vn = _pl_mlp2([usum, umax],
                      [mv[0]["w"][:EMB], mv[0]["w"][EMB:]], mv[0]["b"],
                      mv[1]["w"], mv[1]["b"], add=vn, mn=mn)

    def _head(x, l1, l2, nout):
        w2p = jnp.pad(l2["w"], ((0, 0), (0, EMB - nout)))
        b2p = jnp.pad(l2["b"], (0, EMB - nout))
        y = _pl_mlp2([x], [l1["w"]], l1["b"], w2p, b2p)
        return y[:, :nout]

    out_inst = _head(h_inst, p["fc1_node"], p["fc2_node"], 8)
    out_net = _head(h_net, p["fc1_net"], p["fc2_net"], 8)
    out_vn = _head(vn, p["fc1_vn"], p["fc2_vn"], 8)
    return out_inst, out_net, out_vn


# speedup vs baseline: 1.3115x; 1.0951x over previous
import functools

import jax
import jax.numpy as jnp
from jax import lax
from jax.experimental import pallas as pl
from jax.experimental.pallas import tpu as pltpu
from jax.experimental.pallas import tpu_sc as plsc

N_NODES = 10000
N_NETS = 10000
N_SITES = 10000
EMB = 128
BR = 1000


def _leaky(x):
    return jnp.maximum(x, 0.01 * x)


def _ln(y, g, b):
    m = jnp.mean(y, axis=-1, keepdims=True)
    v = jnp.mean((y - m) ** 2, axis=-1, keepdims=True)
    return (y - m) * jax.lax.rsqrt(v + 1e-5) * g + b


def _dot(a, b):
    return jnp.dot(a, b, preferred_element_type=jnp.float32)


def _mlp2_body(nx, want_min, has_add, has_mn, *refs):
    i = pl.program_id(0)
    idx = 0
    xs = refs[idx:idx + nx]; idx += nx
    w1s = refs[idx:idx + nx]; idx += nx
    b1, w2, b2 = refs[idx:idx + 3]; idx += 3
    add_ref = mn_ref_in = None
    if has_add:
        add_ref = refs[idx]; idx += 1
    if has_mn:
        mn_ref_in = refs[idx]; idx += 1
    out_ref = refs[idx]; idx += 1
    h = _dot(xs[0][...], w1s[0][...])
    for k in range(1, nx):
        xk = xs[k][...]
        if has_mn and k == nx - 1:
            xk = jnp.maximum(xk, mn_ref_in[...])
        h = h + _dot(xk, w1s[k][...])
    h = _leaky(h + b1[...])
    y = _dot(h, w2[...]) + b2[...]
    if has_add:
        y = y + add_ref[...]
    out_ref[...] = y
    if want_min:
        mn_ref = refs[idx]
        @pl.when(i == 0)
        def _():
            mn_ref[...] = jnp.full((1, 1), 3.4e38, jnp.float32)
        blk_min = jnp.min(xs[0][...], keepdims=True).reshape(1, 1)
        mn_ref[...] = jnp.minimum(mn_ref[...], blk_min)


def _pl_mlp2(xs, w1s, b1, w2, b2, add=None, min_of_in0=False, mn=None):
    R = xs[0].shape[0]
    n_out = w2.shape[1]
    grid = (R // BR,)
    in_specs = (
        [pl.BlockSpec((BR, x.shape[1]), lambda i: (i, 0)) for x in xs]
        + [pl.BlockSpec(w.shape, lambda i: (0, 0)) for w in w1s]
        + [pl.BlockSpec((1, b1.shape[0]), lambda i: (0, 0)),
           pl.BlockSpec(w2.shape, lambda i: (0, 0)),
           pl.BlockSpec((1, n_out), lambda i: (0, 0))]
    )
    ops = [xs[k] for k in range(len(xs))] + list(w1s) + [
        b1.reshape(1, -1), w2, b2.reshape(1, -1)]
    if add is not None:
        in_specs.append(pl.BlockSpec((BR, n_out), lambda i: (i, 0)))
        ops.append(add)
    if mn is not None:
        in_specs.append(pl.BlockSpec((1, 1), lambda i: (0, 0)))
        ops.append(mn.reshape(1, 1))
    out_shape = [jax.ShapeDtypeStruct((R, n_out), jnp.float32)]
    out_specs = [pl.BlockSpec((BR, n_out), lambda i: (i, 0))]
    if min_of_in0:
        out_shape.append(jax.ShapeDtypeStruct((1, 1), jnp.float32))
        out_specs.append(pl.BlockSpec((1, 1), lambda i: (0, 0)))
    fn = pl.pallas_call(
        functools.partial(_mlp2_body, len(xs), min_of_in0, add is not None,
                          mn is not None),
        grid=grid, in_specs=in_specs, out_specs=out_specs,
        out_shape=out_shape)
    res = fn(*ops)
    if min_of_in0:
        return res[0], res[1][0, 0]
    return res[0]


def _backvn_body(x1, x2, wa, wb, b1, w2, b2, wphi, bphi, h2_ref, g_ref):
    h = _leaky(_dot(x1[...], wa[...]) + _dot(x2[...], wb[...]) + b1[...])
    h2 = _dot(h, w2[...]) + b2[...]
    h2_ref[...] = h2
    g_ref[...] = _dot(h2, wphi[...]) + bphi[...]


def _pl_backvn(h_inst, vng, wa, wb, b1, w2, b2, wphi, bphi):
    R = h_inst.shape[0]
    grid = (R // BR,)
    specs_x = [pl.BlockSpec((BR, EMB), lambda i: (i, 0))] * 2
    specs_w = [pl.BlockSpec(w.shape, lambda i: (0, 0))
               for w in (wa, wb, b1.reshape(1, -1), w2, b2.reshape(1, -1),
                         wphi, bphi.reshape(1, -1))]
    fn = pl.pallas_call(
        _backvn_body, grid=grid, in_specs=specs_x + specs_w,
        out_specs=[pl.BlockSpec((BR, EMB), lambda i: (i, 0))] * 2,
        out_shape=[jax.ShapeDtypeStruct((R, EMB), jnp.float32)] * 2)
    return fn(h_inst, vng, wa, wb, b1.reshape(1, -1), w2, b2.reshape(1, -1),
              wphi, bphi.reshape(1, -1))


def _psinet_body(x1, x2a, x2b, x3a, x3b, wa, wb, wc, b, g, bln, hnn_ref, ln_ref):
    hnn = (_dot(x1[...], wa[...]) + _dot(x2a[...] + x2b[...], wb[...])
           + _dot(x3a[...] + x3b[...], wc[...]) + b[...])
    hnn_ref[...] = hnn
    ln_ref[...] = _leaky(_ln(hnn, g[...], bln[...]))


def _pl_psinet(h_net, src_msgs, sink_aggs, wa, wb, wc, b, g, bln):
    R = h_net.shape[0]
    grid = (R // BR,)
    specs_x = [pl.BlockSpec((BR, EMB), lambda i: (i, 0))] * 5
    wops = (wa, wb, wc, b.reshape(1, -1), g.reshape(1, -1), bln.reshape(1, -1))
    specs_w = [pl.BlockSpec(w.shape, lambda i: (0, 0)) for w in wops]
    fn = pl.pallas_call(
        _psinet_body, grid=grid, in_specs=specs_x + specs_w,
        out_specs=[pl.BlockSpec((BR, EMB), lambda i: (i, 0))] * 2,
        out_shape=[jax.ShapeDtypeStruct((R, EMB), jnp.float32)] * 2)
    return fn(h_net, src_msgs[0], src_msgs[1], sink_aggs[0], sink_aggs[1], *wops)


def _psinode_body(x1, x2, x2b, wa, wb, b, g, bln, out_ref, mn_ref):
    i = pl.program_id(0)
    hin = _dot(x1[...], wa[...]) + _dot(x2[...] + x2b[...], wb[...]) + b[...]
    y = _leaky(_ln(hin, g[...], bln[...]))
    out_ref[...] = y
    @pl.when(i == 0)
    def _():
        mn_ref[...] = jnp.full((1, 1), 3.4e38, jnp.float32)
    mn_ref[...] = jnp.minimum(mn_ref[...], jnp.min(y, keepdims=True).reshape(1, 1))


def _pl_psinode(h2, back, back_b, wa, wb, b, g, bln):
    R = h2.shape[0]
    grid = (R // BR,)
    specs_x = [pl.BlockSpec((BR, EMB), lambda i: (i, 0))] * 3
    wops = (wa, wb, b.reshape(1, -1), g.reshape(1, -1), bln.reshape(1, -1))
    specs_w = [pl.BlockSpec(w.shape, lambda i: (0, 0)) for w in wops]
    fn = pl.pallas_call(
        _psinode_body, grid=grid, in_specs=specs_x + specs_w,
        out_specs=[pl.BlockSpec((BR, EMB), lambda i: (i, 0)),
                   pl.BlockSpec((1, 1), lambda i: (0, 0))],
        out_shape=[jax.ShapeDtypeStruct((R, EMB), jnp.float32),
                   jax.ShapeDtypeStruct((1, 1), jnp.float32)])
    out, mn = fn(h2, back, back_b, *wops)
    return out, mn[0, 0]


def _eproj_body(attr, we, out_ref):
    a = attr[...]
    w = we[...]
    acc = a[:, 0:1] * w[0:1, :]
    for k in range(1, 4):
        acc = acc + a[:, k:k + 1] * w[k:k + 1, :]
    out_ref[...] = acc


def _pl_eproj(attr_pad, we):
    R = attr_pad.shape[0]
    BRE = 2048
    fn = pl.pallas_call(
        _eproj_body, grid=(R // BRE,),
        in_specs=[pl.BlockSpec((BRE, 4), lambda i: (i, 0)),
                  pl.BlockSpec((4, EMB), lambda i: (0, 0))],
        out_specs=pl.BlockSpec((BRE, EMB), lambda i: (i, 0)),
        out_shape=jax.ShapeDtypeStruct((R, EMB), jnp.float32))
    return fn(attr_pad, we)


ACC_ROWS = 10240


def _sc_scatter_sum(table, gath2d, scat2d, zeros):
    nchunk, cw = gath2d.shape
    per_w = nchunk // 32
    mesh = plsc.VectorSubcoreMesh(core_axis_name="c", subcore_axis_name="s")

    @functools.partial(
        pl.kernel, mesh=mesh,
        out_type=jax.ShapeDtypeStruct((2, ACC_ROWS, EMB), jnp.float32),
        scratch_types=[
            pltpu.VMEM((1, cw), jnp.int32),
            pltpu.VMEM((1, cw), jnp.int32),
            pltpu.VMEM((1, cw), jnp.int32),
            pltpu.VMEM((1, cw), jnp.int32),
            pltpu.VMEM((cw, EMB), jnp.float32),
            pltpu.VMEM((cw, EMB), jnp.float32),
            pltpu.VMEM_SHARED((ACC_ROWS, EMB), jnp.float32),
            pltpu.SemaphoreType.DMA,
            pltpu.SemaphoreType.DMA,
            pltpu.SemaphoreType.DMA,
            pltpu.SemaphoreType.DMA,
            pltpu.SemaphoreType.DMA,
            pltpu.SemaphoreType.DMA,
        ])
    def k(table_h, gath_h, scat_h, zeros_h, out_h, gi0, gi1, si0, si1,
          rows0, rows1, acc, se0, se1, sg0, sg1, ss0, ss1):
        c = lax.axis_index("c")
        s = lax.axis_index("s")

        @pl.when(s == 0)
        def _():
            pltpu.sync_copy(zeros_h, acc)
        base = (s * 2 + c) * per_w
        plsc.subcore_barrier()

        gi = (gi0, gi1)
        si = (si0, si1)
        rows = (rows0, rows1)
        se = (se0, se1)
        sg = (sg0, sg1)
        ss = (ss0, ss1)

        def body(i, carry):
            hi = [None, None]
            hg = [None, None]
            hs = [None, None]
            for b in range(2):
                ch = base + i * 2 + b
                hi[b] = (pltpu.async_copy(gath_h.at[pl.ds(ch, 1)], gi[b],
                                          se[b]),
                         pltpu.async_copy(scat_h.at[pl.ds(ch, 1)], si[b],
                                          se[b]))
            for b in range(2):
                hi[b][0].wait()
                hi[b][1].wait()
                hg[b] = pltpu.async_copy(table_h.at[gi[b].at[0]], rows[b],
                                         sg[b])
            for b in range(2):
                hg[b].wait()
                hs[b] = pltpu.async_copy(rows[b], acc.at[si[b].at[0]],
                                         ss[b], add=True)
            for b in range(2):
                hs[b].wait()
            return carry

        lax.fori_loop(0, per_w // 2, body, 0)
        plsc.subcore_barrier()
        stripe = ACC_ROWS // 16
        pltpu.sync_copy(acc.at[pl.ds(s * stripe, stripe)],
                        out_h.at[c].at[pl.ds(s * stripe, stripe)])

    return k(table, gath2d, scat2d, zeros)


def _sc_sink_src(g, h2, ep, sk_g2d, sk_s2d, sr_g2d, sr_s2d, zeros):
    n_sink = sk_g2d.shape[0] // 32
    n_src = sr_g2d.shape[0] // 32
    mesh = plsc.VectorSubcoreMesh(core_axis_name="c", subcore_axis_name="s")

    @functools.partial(
        pl.kernel, mesh=mesh,
        out_type=[jax.ShapeDtypeStruct((2, ACC_ROWS, EMB), jnp.float32),
                  jax.ShapeDtypeStruct((2, ACC_ROWS, EMB), jnp.float32)],
        scratch_types=[
            pltpu.VMEM((1, 128), jnp.int32),
            pltpu.VMEM((1, 128), jnp.int32),
            pltpu.VMEM((1, 128), jnp.int32),
            pltpu.VMEM((1, 128), jnp.int32),
            pltpu.VMEM((128, EMB), jnp.float32),
            pltpu.VMEM((128, EMB), jnp.float32),
            pltpu.VMEM_SHARED((ACC_ROWS, EMB), jnp.float32),
            pltpu.SemaphoreType.DMA,
            pltpu.SemaphoreType.DMA,
            pltpu.SemaphoreType.DMA,
            pltpu.SemaphoreType.DMA,
            pltpu.SemaphoreType.DMA,
            pltpu.SemaphoreType.DMA,
            pltpu.SemaphoreType.DMA,
            pltpu.SemaphoreType.DMA,
        ])
    def k(g_h, h2_h, ep_h, skg_h, sks_h, srg_h, srs_h, zeros_h,
          sink_o, src_o, gi0, gi1, si0, si1, rows0, rows1, acc,
          si_s0, si_s1, sp0, sp1, sg0, sg1, ss0, ss1):
        c = lax.axis_index("c")
        s = lax.axis_index("s")
        stripe = ACC_ROWS // 16
        gi = (gi0, gi1)
        si = (si0, si1)
        rows = (rows0, rows1)
        sei = (si_s0, si_s1)
        sep = (sp0, sp1)
        sg = (sg0, sg1)
        ss = (ss0, ss1)

        def zero_acc():
            @pl.when(s == 0)
            def _():
                pltpu.sync_copy(zeros_h, acc)

        zero_acc()
        base = (s * 2 + c) * n_sink
        plsc.subcore_barrier()

        def leaky_buf(rbuf):
            def leaky_rows(r, carry):
                for j in range(EMB // 16):
                    v = rbuf[r, pl.ds(j * 16, 16)]
                    rbuf[r, pl.ds(j * 16, 16)] = jnp.maximum(v, 0.01 * v)
                return carry
            lax.fori_loop(0, 128, leaky_rows, 0)

        def sink_body(i, carry):
            hi = [None, None]
            hp = [None, None]
            hg = [None, None]
            hs = [None, None]
            for b in range(2):
                ch = base + i * 2 + b
                hi[b] = (pltpu.async_copy(skg_h.at[pl.ds(ch, 1)], gi[b],
                                          sei[b]),
                         pltpu.async_copy(sks_h.at[pl.ds(ch, 1)], si[b],
                                          sei[b]))
                hp[b] = pltpu.async_copy(
                    ep_h.at[pl.ds(pl.multiple_of(ch * 128, 128), 128)],
                    rows[b], sep[b])
            for b in range(2):
                hi[b][0].wait()
                hi[b][1].wait()
                hp[b].wait()
                hg[b] = pltpu.async_copy(g_h.at[gi[b].at[0]], rows[b],
                                         sg[b], add=True)
            for b in range(2):
                hg[b].wait()
                leaky_buf(rows[b])
                hs[b] = pltpu.async_copy(rows[b], acc.at[si[b].at[0]],
                                         ss[b], add=True)
            for b in range(2):
                hs[b].wait()
            return carry

        lax.fori_loop(0, n_sink // 2, sink_body, 0)
        plsc.subcore_barrier()
        pltpu.sync_copy(acc.at[pl.ds(s * stripe, stripe)],
                        sink_o.at[c].at[pl.ds(s * stripe, stripe)])
        plsc.subcore_barrier()
        zero_acc()
        base2 = (s * 2 + c) * n_src
        plsc.subcore_barrier()

        def src_body(i, carry):
            ch = base2 + i
            pltpu.sync_copy(srg_h.at[pl.ds(ch, 1)], gi0)
            pltpu.sync_copy(srs_h.at[pl.ds(ch, 1)], si0)
            pltpu.async_copy(h2_h.at[gi0.at[0]], rows0, sg0)
            pltpu.make_async_copy(h2_h.at[gi0.at[0]], rows0, sg0).wait()
            pltpu.sync_copy(rows0, acc.at[si0.at[0]], add=True)
            return carry

        lax.fori_loop(0, n_src, src_body, 0)
        plsc.subcore_barrier()
        pltpu.sync_copy(acc.at[pl.ds(s * stripe, stripe)],
                        src_o.at[c].at[pl.ds(s * stripe, stripe)])

    return k(g, h2, ep, sk_g2d, sk_s2d, sr_g2d, sr_s2d, zeros)


def _sc_gather_rows(table, idx_pad):
    n = idx_pad.shape[0]
    per_w = n // 32
    mesh = plsc.VectorSubcoreMesh(core_axis_name="c", subcore_axis_name="s")

    @functools.partial(
        pl.kernel, mesh=mesh,
        out_type=jax.ShapeDtypeStruct((n, EMB), jnp.float32),
        scratch_types=[
            pltpu.VMEM((per_w,), jnp.int32),
            pltpu.VMEM((per_w, EMB), jnp.float32),
            pltpu.SemaphoreType.DMA,
        ])
    def k(table_h, idx_h, out_h, idxv, rows, sem):
        c = lax.axis_index("c")
        s = lax.axis_index("s")
        base = pl.multiple_of((s * 2 + c) * per_w, 8)
        pltpu.sync_copy(idx_h.at[pl.ds(base, per_w)], idxv)
        pltpu.async_copy(table_h.at[idxv], rows, sem).wait()
        pltpu.sync_copy(rows, out_h.at[pl.ds(base, per_w)])

    return k(table, idx_pad)


def _pad_edges_2d(idx, n_pad_to, dummy_base, cw=128):
    extra = n_pad_to - idx.shape[0]
    if dummy_base is None:
        pad = (jnp.arange(extra, dtype=jnp.int32) * 17) % N_NODES
    else:
        pad = dummy_base + (jnp.arange(extra, dtype=jnp.int32) % 200)
    return jnp.concatenate([idx, pad]).reshape(-1, cw)


def kernel(node_x, net_x, edge_attr_sink_to_net, params,
           edge_index_source_to_net, edge_index_sink_to_net, pos_lst):
    p = params
    idx = pos_lst[:, 0].astype(jnp.int32) * 100 + pos_lst[:, 1].astype(jnp.int32)

    idx_pad = jnp.concatenate([idx, jnp.zeros((240,), jnp.int32)])

    NPAD = 10240
    ugath2d = jnp.minimum(jnp.arange(NPAD, dtype=jnp.int32),
                          N_NODES - 1).reshape(-1, 80)
    uscat2d = _pad_edges_2d(idx, NPAD, N_SITES, cw=80)
    zeros_acc0 = jnp.zeros((ACC_ROWS, EMB), jnp.float32)

    def _usum(feat):
        part = _sc_scatter_sum(feat, ugath2d, uscat2d, zeros_acc0)
        return part[0, :N_SITES], part[1, :N_SITES]

    us0, us1 = _usum(node_x)
    umax = jnp.full((N_SITES, EMB), -3.4e38, jnp.float32).at[idx].max(node_x)

    ne = p["node_enc"]
    h_inst, mn0 = _pl_mlp2([node_x], [ne[0]["w"]], ne[0]["b"],
                           ne[1]["w"], ne[1]["b"], min_of_in0=True)
    ve = p["vn_enc"]
    vn = _pl_mlp2([us0, us1, umax],
                  [ve[0]["w"][:EMB], ve[0]["w"][:EMB], ve[0]["w"][EMB:]],
                  ve[0]["b"], ve[1]["w"], ve[1]["b"], mn=mn0)
    te = p["net_enc"]
    h_net = _pl_mlp2([net_x], [te[0]["w"]], te[0]["b"], te[1]["w"], te[1]["b"])

    sink_src = edge_index_sink_to_net[0]
    sink_dst = edge_index_sink_to_net[1]
    src_src = edge_index_source_to_net[0]
    src_dst = edge_index_source_to_net[1]

    EP = 327680
    EPS = 12288
    sink_dst2d = _pad_edges_2d(sink_dst, EP, None)
    sink_src2d_scat = _pad_edges_2d(sink_src, EP, N_NODES)
    sink_src2d_g = _pad_edges_2d(sink_src, EP, None)
    sink_dst2d_scat = _pad_edges_2d(sink_dst, EP, N_NETS)
    src_src2d_g = _pad_edges_2d(src_src, EPS, None)
    src_dst2d_scat = _pad_edges_2d(src_dst, EPS, N_NETS)
    attr_pad = jnp.pad(edge_attr_sink_to_net, ((0, EP - sink_src.shape[0]), (0, 0)))
    zeros_acc = jnp.zeros((ACC_ROWS, EMB), jnp.float32)

    for lp in p["layers"]:
        vng = _sc_gather_rows(vn, idx_pad)[:N_NODES]
        bv = lp["back_vn"]
        h2, g = _pl_backvn(h_inst, vng,
                           bv[0]["w"][:EMB], bv[0]["w"][EMB:], bv[0]["b"],
                           bv[1]["w"], bv[1]["b"],
                           lp["phi"]["w"][:EMB], lp["phi"]["b"])
        eproj = _pl_eproj(attr_pad, lp["phi"]["w"][EMB:])
        sinkp, srcp = _sc_sink_src(g, h2, eproj,
                                   sink_src2d_g, sink_dst2d_scat,
                                   src_src2d_g, src_dst2d_scat, zeros_acc)

        pn = lp["psi_net"]
        hnn, h_net = _pl_psinet(h_net,
                                (srcp[0, :N_NETS], srcp[1, :N_NETS]),
                                (sinkp[0, :N_NETS], sinkp[1, :N_NETS]),
                                pn["w"][:EMB], pn["w"][EMB:2 * EMB],
                                pn["w"][2 * EMB:], pn["b"],
                                lp["norm_g"], lp["norm_b"])
        backp = _sc_scatter_sum(hnn, sink_dst2d, sink_src2d_scat, zeros_acc)
        pd = lp["psi_node"]
        h_inst, mn = _pl_psinode(h2, backp[0, :N_NODES], backp[1, :N_NODES],
                                 pd["w"][:EMB], pd["w"][EMB:],
                                 pd["b"], lp["norm_g"], lp["norm_b"])

        us0, us1 = _usum(h_inst)
        umax = jnp.full((N_SITES, EMB), -3.4e38, jnp.float32).at[idx].max(h_inst)
        mv = lp["mlp_vn"]
        vn = _pl_mlp2([us0, us1, umax],
                      [mv[0]["w"][:EMB], mv[0]["w"][:EMB], mv[0]["w"][EMB:]],
                      mv[0]["b"], mv[1]["w"], mv[1]["b"], add=vn, mn=mn)

    def _head(x, l1, l2, nout):
        w2p = jnp.pad(l2["w"], ((0, 0), (0, EMB - nout)))
        b2p = jnp.pad(l2["b"], (0, EMB - nout))
        y = _pl_mlp2([x], [l1["w"]], l1["b"], w2p, b2p)
        return y[:, :nout]

    out_inst = _head(h_inst, p["fc1_node"], p["fc2_node"], 8)
    out_net = _head(h_net, p["fc1_net"], p["fc2_net"], 8)
    out_vn = _head(vn, p["fc1_vn"], p["fc2_vn"], 8)
    return out_inst, out_net, out_vn
